# baseline (speedup 1.0000x reference)
import functools

import jax
import jax.numpy as jnp
import numpy as np
from jax import lax
from jax.experimental import pallas as pl
from jax.experimental.pallas import tpu as pltpu
from jax.experimental.pallas import tpu_sc as plsc

H = 768
HP = H // 2
NTOK = 8192
NLBL = 64

SPLIT = 2048

NC = 2
NS = 16
NW = NC * NS
PER_W = SPLIT // NW

BT = 2048
NB_SC = SPLIT // BT
NB_OH = (NTOK - SPLIT) // BT


def _I0():
    return jnp.int32(0)


_M16 = np.int32(0xFFFF)


def _pack_bf16_pair(a, b):
    lo = lax.bitcast_convert_type(a, jnp.int16).astype(jnp.int32) & _M16
    hi = lax.bitcast_convert_type(b, jnp.int16).astype(jnp.int32) << 16
    return hi | lo


def _unpack_bf16_pair(p):
    lo = lax.bitcast_convert_type((p & _M16).astype(jnp.int16), jnp.bfloat16)
    hi = lax.bitcast_convert_type((p >> 16).astype(jnp.int16), jnp.bfloat16)
    return lo, hi


def _table_body(lf_ref, w2_ref, b_ref, tp_ref):
    t = (
        jnp.dot(lf_ref[...], w2_ref[...], preferred_element_type=jnp.float32)
        + b_ref[...].reshape(1, H)
    ).astype(jnp.bfloat16)
    tp_ref[...] = _pack_bf16_pair(t[:, :HP], t[:, HP:])


def _project_table(label_feature, w_merge, b2d):
    return pl.pallas_call(
        _table_body,
        grid=(1,),
        in_specs=[
            pl.BlockSpec((NLBL, H), lambda i: (_I0(), _I0())),
            pl.BlockSpec((H, H), lambda i: (jnp.int32(1), _I0())),
            pl.BlockSpec((H,), lambda i: (_I0(),)),
        ],
        out_specs=pl.BlockSpec((NLBL, HP), lambda i: (_I0(), _I0())),
        out_shape=jax.ShapeDtypeStruct((NLBL, HP), jnp.int32),
    )(label_feature, w_merge, b2d)


@functools.cache
def _make_sc_gather():
    @functools.partial(
        pl.kernel,
        out_type=jax.ShapeDtypeStruct((SPLIT, HP), jnp.int32),
        mesh=plsc.VectorSubcoreMesh(
            core_axis_name="c",
            subcore_axis_name="s",
            num_cores=NC,
            num_subcores=NS,
        ),
        scratch_types=[
            pltpu.VMEM((PER_W,), jnp.int32),
            pltpu.VMEM((PER_W, HP), jnp.int32),
            pltpu.SemaphoreType.DMA,
        ],
    )
    def _sc_gather(lbl_hbm, t_hbm, g_hbm, idx_v, rows_v, sem):
        wid = lax.axis_index("s") * NC + lax.axis_index("c")
        base = wid * PER_W
        pltpu.sync_copy(
            lbl_hbm.at[np.int32(0), np.int32(0), pl.ds(base, PER_W)], idx_v
        )
        for j in range(PER_W // 16):
            v = idx_v[pl.ds(j * 16, 16)]
            idx_v[pl.ds(j * 16, 16)] = jnp.maximum(v - 1, 0)
        pltpu.async_copy(t_hbm.at[idx_v], rows_v, sem).wait()
        pltpu.sync_copy(rows_v, g_hbm.at[pl.ds(base, PER_W)])

    return _sc_gather


def _merge_oh_body(x_ref, l_ref, lf_ref, w2_ref, b_ref, w1_ref, o_ref, t_s):
    @pl.when(pl.program_id(0) == 0)
    def _compute_table():
        t_s[...] = (
            jnp.dot(
                lf_ref[...], w2_ref[...], preferred_element_type=jnp.float32
            )
            + b_ref[...].reshape(1, H)
        ).astype(jnp.bfloat16)

    x = x_ref[...]
    lblr = l_ref[0]
    y = jnp.dot(
        x.astype(jnp.bfloat16), w1_ref[...].astype(jnp.bfloat16),
        preferred_element_type=jnp.float32,
    )
    oht = (
        lax.broadcasted_iota(jnp.int32, (NLBL, BT), 0) == (lblr - 1)
    ).astype(jnp.bfloat16)
    g = lax.dot_general(
        oht, t_s[...], (((0,), (0,)), ((), ())),
        preferred_element_type=jnp.float32,
    )
    m = lax.dot_general(
        oht, jnp.ones((NLBL, 1), jnp.bfloat16), (((0,), (0,)), ((), ())),
        preferred_element_type=jnp.float32,
    )
    o_ref[...] = x + m * (y + g - x)


def _merge_gather_body(x_ref, l_ref, g_ref, w1_ref, prev_ref, o_ref):
    del prev_ref
    x = x_ref[...]
    lblr = l_ref[0]
    y = jnp.dot(
        x.astype(jnp.bfloat16), w1_ref[...].astype(jnp.bfloat16),
        preferred_element_type=jnp.float32,
    )
    glo, ghi = _unpack_bf16_pair(g_ref[...])
    g = jnp.concatenate([glo, ghi], axis=1).astype(jnp.float32)
    mrow = (lblr != 0).astype(jnp.bfloat16)
    m = lax.dot_general(
        mrow, jnp.ones((1, 1), jnp.bfloat16), (((0,), (0,)), ((), ())),
        preferred_element_type=jnp.float32,
    )
    o_ref[...] = x + m * (y + g - x)


_OUT_SHAPE = jax.ShapeDtypeStruct((NTOK, H), jnp.float32)


def _NB_OFF():
    return jnp.int32(NB_SC)


def _merge_onehot(x, lbl2d, label_feature, b2d, w_merge):
    return pl.pallas_call(
        _merge_oh_body,
        grid=(NB_OH,),
        in_specs=[
            pl.BlockSpec((BT, H), lambda i: (i + _NB_OFF(), _I0())),
            pl.BlockSpec((1, 1, BT), lambda i: (i + _NB_OFF(), _I0(), _I0())),
            pl.BlockSpec((NLBL, H), lambda i: (_I0(), _I0())),
            pl.BlockSpec((H, H), lambda i: (jnp.int32(1), _I0())),
            pl.BlockSpec((H,), lambda i: (_I0(),)),
            pl.BlockSpec((H, H), lambda i: (_I0(), _I0())),
        ],
        out_specs=pl.BlockSpec((BT, H), lambda i: (i + _NB_OFF(), _I0())),
        out_shape=_OUT_SHAPE,
        scratch_shapes=[pltpu.VMEM((NLBL, H), jnp.bfloat16)],
        compiler_params=pltpu.CompilerParams(
            dimension_semantics=("parallel",)
        ),
    )(x, lbl2d, label_feature, w_merge, b2d, w_merge)


def _merge_gather(x, lbl2d, g, w_merge, prev):
    return pl.pallas_call(
        _merge_gather_body,
        grid=(NB_SC,),
        in_specs=[
            pl.BlockSpec((BT, H), lambda i: (i, _I0())),
            pl.BlockSpec((1, 1, BT), lambda i: (i, _I0(), _I0())),
            pl.BlockSpec((BT, HP), lambda i: (i, _I0())),
            pl.BlockSpec((H, H), lambda i: (_I0(), _I0())),
            pl.BlockSpec((8, 128), lambda i: (_I0(), _I0())),
        ],
        out_specs=pl.BlockSpec((BT, H), lambda i: (i, _I0())),
        out_shape=_OUT_SHAPE,
        input_output_aliases={4: 0},
        compiler_params=pltpu.CompilerParams(
            dimension_semantics=("parallel",)
        ),
    )(x, lbl2d, g, w_merge, prev)


def kernel(com_features, labels, label_feature, W_merge, b_merge):
    x = com_features.reshape(NTOK, H)
    lbl3d = labels.reshape(NTOK // BT, 1, BT).astype(jnp.int32)
    t_packed = _project_table(label_feature, W_merge, b_merge)
    g = _make_sc_gather()(lbl3d, t_packed)
    out = _merge_onehot(x, lbl3d, label_feature, b_merge, W_merge)
    out = _merge_gather(x, lbl3d, g, W_merge, out)
    return out.reshape(com_features.shape)


# speedup vs baseline: 1.0600x; 1.0600x over previous
import functools

import jax
import jax.numpy as jnp
import numpy as np
from jax import lax
from jax.experimental import pallas as pl
from jax.experimental.pallas import tpu as pltpu
from jax.experimental.pallas import tpu_sc as plsc

H = 768
HP = H // 2
NTOK = 8192
NLBL = 64

SPLIT = 1024

NC = 2
NS = 16
NW = NC * NS
PER_W = SPLIT // NW

BT = 1024
NB_SC = SPLIT // BT
NB_OH = (NTOK - SPLIT) // BT


def _I0():
    return jnp.int32(0)


_M16 = np.int32(0xFFFF)


def _pack_bf16_pair(a, b):
    lo = lax.bitcast_convert_type(a, jnp.int16).astype(jnp.int32) & _M16
    hi = lax.bitcast_convert_type(b, jnp.int16).astype(jnp.int32) << 16
    return hi | lo


def _unpack_bf16_pair(p):
    lo = lax.bitcast_convert_type((p & _M16).astype(jnp.int16), jnp.bfloat16)
    hi = lax.bitcast_convert_type((p >> 16).astype(jnp.int16), jnp.bfloat16)
    return lo, hi


def _table_body(lf_ref, w2_ref, b_ref, tp_ref):
    t = (
        jnp.dot(lf_ref[...], w2_ref[...], preferred_element_type=jnp.float32)
        + b_ref[...].reshape(1, H)
    ).astype(jnp.bfloat16)
    tp_ref[...] = _pack_bf16_pair(t[:, :HP], t[:, HP:])


def _project_table(label_feature, w_merge, b2d):
    return pl.pallas_call(
        _table_body,
        grid=(1,),
        in_specs=[
            pl.BlockSpec((NLBL, H), lambda i: (_I0(), _I0())),
            pl.BlockSpec((H, H), lambda i: (jnp.int32(1), _I0())),
            pl.BlockSpec((H,), lambda i: (_I0(),)),
        ],
        out_specs=pl.BlockSpec((NLBL, HP), lambda i: (_I0(), _I0())),
        out_shape=jax.ShapeDtypeStruct((NLBL, HP), jnp.int32),
    )(label_feature, w_merge, b2d)


@functools.cache
def _make_sc_gather():
    @functools.partial(
        pl.kernel,
        out_type=jax.ShapeDtypeStruct((SPLIT, HP), jnp.int32),
        mesh=plsc.VectorSubcoreMesh(
            core_axis_name="c",
            subcore_axis_name="s",
            num_cores=NC,
            num_subcores=NS,
        ),
        scratch_types=[
            pltpu.VMEM((PER_W,), jnp.int32),
            pltpu.VMEM((PER_W, HP), jnp.int32),
            pltpu.SemaphoreType.DMA,
        ],
    )
    def _sc_gather(lbl_hbm, t_hbm, g_hbm, idx_v, rows_v, sem):
        wid = lax.axis_index("s") * NC + lax.axis_index("c")
        base = wid * PER_W
        pltpu.sync_copy(
            lbl_hbm.at[np.int32(0), np.int32(0), pl.ds(base, PER_W)], idx_v
        )
        for j in range(PER_W // 16):
            v = idx_v[pl.ds(j * 16, 16)]
            idx_v[pl.ds(j * 16, 16)] = jnp.maximum(v - 1, 0)
        pltpu.async_copy(t_hbm.at[idx_v], rows_v, sem).wait()
        pltpu.sync_copy(rows_v, g_hbm.at[pl.ds(base, PER_W)])

    return _sc_gather


def _merge_oh_body(x_ref, l_ref, lf_ref, w2_ref, b_ref, w1_ref, o_ref, t_s):
    @pl.when(pl.program_id(0) == 0)
    def _compute_table():
        t_s[...] = (
            jnp.dot(
                lf_ref[...], w2_ref[...], preferred_element_type=jnp.float32
            )
            + b_ref[...].reshape(1, H)
        ).astype(jnp.bfloat16)

    x = x_ref[...]
    lblr = l_ref[0]
    y = jnp.dot(
        x.astype(jnp.bfloat16), w1_ref[...].astype(jnp.bfloat16),
        preferred_element_type=jnp.float32,
    )
    oht = (
        lax.broadcasted_iota(jnp.int32, (NLBL, BT), 0) == (lblr - 1)
    ).astype(jnp.bfloat16)
    g = lax.dot_general(
        oht, t_s[...], (((0,), (0,)), ((), ())),
        preferred_element_type=jnp.float32,
    )
    m = lax.dot_general(
        oht, jnp.ones((NLBL, 1), jnp.bfloat16), (((0,), (0,)), ((), ())),
        preferred_element_type=jnp.float32,
    )
    o_ref[...] = x + m * (y + g - x)


def _merge_gather_body(x_ref, l_ref, g_ref, w1_ref, prev_ref, o_ref):
    del prev_ref
    x = x_ref[...]
    lblr = l_ref[0]
    y = jnp.dot(
        x.astype(jnp.bfloat16), w1_ref[...].astype(jnp.bfloat16),
        preferred_element_type=jnp.float32,
    )
    glo, ghi = _unpack_bf16_pair(g_ref[...])
    g = jnp.concatenate([glo, ghi], axis=1).astype(jnp.float32)
    mrow = (lblr != 0).astype(jnp.bfloat16)
    m = lax.dot_general(
        mrow, jnp.ones((1, 1), jnp.bfloat16), (((0,), (0,)), ((), ())),
        preferred_element_type=jnp.float32,
    )
    o_ref[...] = x + m * (y + g - x)


_OUT_SHAPE = jax.ShapeDtypeStruct((NTOK, H), jnp.float32)


def _NB_OFF():
    return jnp.int32(NB_SC)


def _merge_onehot(x, lbl2d, label_feature, b2d, w_merge):
    return pl.pallas_call(
        _merge_oh_body,
        grid=(NB_OH,),
        in_specs=[
            pl.BlockSpec((BT, H), lambda i: (i + _NB_OFF(), _I0())),
            pl.BlockSpec((1, 1, BT), lambda i: (i + _NB_OFF(), _I0(), _I0())),
            pl.BlockSpec((NLBL, H), lambda i: (_I0(), _I0())),
            pl.BlockSpec((H, H), lambda i: (jnp.int32(1), _I0())),
            pl.BlockSpec((H,), lambda i: (_I0(),)),
            pl.BlockSpec((H, H), lambda i: (_I0(), _I0())),
        ],
        out_specs=pl.BlockSpec((BT, H), lambda i: (i + _NB_OFF(), _I0())),
        out_shape=_OUT_SHAPE,
        scratch_shapes=[pltpu.VMEM((NLBL, H), jnp.bfloat16)],
        compiler_params=pltpu.CompilerParams(
            dimension_semantics=("parallel",)
        ),
    )(x, lbl2d, label_feature, w_merge, b2d, w_merge)


def _merge_gather(x, lbl2d, g, w_merge, prev):
    return pl.pallas_call(
        _merge_gather_body,
        grid=(NB_SC,),
        in_specs=[
            pl.BlockSpec((BT, H), lambda i: (i, _I0())),
            pl.BlockSpec((1, 1, BT), lambda i: (i, _I0(), _I0())),
            pl.BlockSpec((BT, HP), lambda i: (i, _I0())),
            pl.BlockSpec((H, H), lambda i: (_I0(), _I0())),
            pl.BlockSpec((8, 128), lambda i: (_I0(), _I0())),
        ],
        out_specs=pl.BlockSpec((BT, H), lambda i: (i, _I0())),
        out_shape=_OUT_SHAPE,
        input_output_aliases={4: 0},
        compiler_params=pltpu.CompilerParams(
            dimension_semantics=("parallel",)
        ),
    )(x, lbl2d, g, w_merge, prev)


def kernel(com_features, labels, label_feature, W_merge, b_merge):
    x = com_features.reshape(NTOK, H)
    lbl3d = labels.reshape(NTOK // BT, 1, BT).astype(jnp.int32)
    t_packed = _project_table(label_feature, W_merge, b_merge)
    g = _make_sc_gather()(lbl3d, t_packed)
    out = _merge_onehot(x, lbl3d, label_feature, b_merge, W_merge)
    out = _merge_gather(x, lbl3d, g, W_merge, out)
    return out.reshape(com_features.shape)


# speedup vs baseline: 1.0746x; 1.0138x over previous
import functools

import jax
import jax.numpy as jnp
import numpy as np
from jax import lax
from jax.experimental import pallas as pl
from jax.experimental.pallas import tpu as pltpu
from jax.experimental.pallas import tpu_sc as plsc

H = 768
HP = H // 2
NTOK = 8192
NLBL = 64

SPLIT = 1024

NC = 2
NS = 16
NW = NC * NS
PER_W = SPLIT // NW

BT = 1024
NB_SC = SPLIT // BT
NB_OH = (NTOK - SPLIT) // BT


def _I0():
    return jnp.int32(0)


_M16 = np.int32(0xFFFF)


def _pack_bf16_pair(a, b):
    lo = lax.bitcast_convert_type(a, jnp.int16).astype(jnp.int32) & _M16
    hi = lax.bitcast_convert_type(b, jnp.int16).astype(jnp.int32) << 16
    return hi | lo


def _unpack_bf16_pair(p):
    lo = lax.bitcast_convert_type((p & _M16).astype(jnp.int16), jnp.bfloat16)
    hi = lax.bitcast_convert_type((p >> 16).astype(jnp.int16), jnp.bfloat16)
    return lo, hi


def _table_body(lf_ref, w2_ref, b_ref, tp_ref):
    t = (
        jnp.dot(lf_ref[...], w2_ref[...], preferred_element_type=jnp.float32)
        + b_ref[...].reshape(1, H)
    ).astype(jnp.bfloat16)
    tp_ref[...] = _pack_bf16_pair(t[:, :HP], t[:, HP:])


def _project_table(label_feature, w_merge, b2d):
    return pl.pallas_call(
        _table_body,
        grid=(1,),
        in_specs=[
            pl.BlockSpec((NLBL, H), lambda i: (_I0(), _I0())),
            pl.BlockSpec((H, H), lambda i: (jnp.int32(1), _I0())),
            pl.BlockSpec((H,), lambda i: (_I0(),)),
        ],
        out_specs=pl.BlockSpec((NLBL, HP), lambda i: (_I0(), _I0())),
        out_shape=jax.ShapeDtypeStruct((NLBL, HP), jnp.int32),
    )(label_feature, w_merge, b2d)


@functools.cache
def _make_sc_gather():
    @functools.partial(
        pl.kernel,
        out_type=jax.ShapeDtypeStruct((SPLIT, HP), jnp.int32),
        mesh=plsc.VectorSubcoreMesh(
            core_axis_name="c",
            subcore_axis_name="s",
            num_cores=NC,
            num_subcores=NS,
        ),
        scratch_types=[
            pltpu.VMEM((PER_W,), jnp.int32),
            pltpu.VMEM((PER_W, HP), jnp.int32),
            pltpu.SemaphoreType.DMA,
        ],
    )
    def _sc_gather(lbl_hbm, t_hbm, g_hbm, idx_v, rows_v, sem):
        wid = lax.axis_index("s") * NC + lax.axis_index("c")
        base = wid * PER_W
        pltpu.sync_copy(
            lbl_hbm.at[np.int32(0), np.int32(0), pl.ds(base, PER_W)], idx_v
        )
        for j in range(PER_W // 16):
            v = idx_v[pl.ds(j * 16, 16)]
            idx_v[pl.ds(j * 16, 16)] = jnp.maximum(v - 1, 0)
        pltpu.async_copy(t_hbm.at[idx_v], rows_v, sem).wait()
        pltpu.sync_copy(rows_v, g_hbm.at[pl.ds(base, PER_W)])

    return _sc_gather


def _merge_oh_body(x_ref, l_ref, lf_ref, w2_ref, b_ref, w1_ref, o_ref, t_s):
    @pl.when(pl.program_id(0) == 0)
    def _compute_table():
        t_s[...] = (
            jnp.dot(
                lf_ref[...], w2_ref[...], preferred_element_type=jnp.float32
            )
            + b_ref[...].reshape(1, H)
        ).astype(jnp.bfloat16)

    x = x_ref[...]
    lblr = l_ref[0]
    y = jnp.dot(
        x.astype(jnp.bfloat16), w1_ref[...].astype(jnp.bfloat16),
        preferred_element_type=jnp.float32,
    )
    oht = (
        lax.broadcasted_iota(jnp.int32, (NLBL, BT), 0) == (lblr - 1)
    ).astype(jnp.bfloat16)
    g = lax.dot_general(
        oht, t_s[...], (((0,), (0,)), ((), ())),
        preferred_element_type=jnp.float32,
    )
    m = lax.dot_general(
        oht, jnp.ones((NLBL, 1), jnp.bfloat16), (((0,), (0,)), ((), ())),
        preferred_element_type=jnp.float32,
    )
    o_ref[...] = x + m * (y + g - x)


def _merge_gather_body(x_ref, l_ref, g_ref, w1_ref, prev_ref, o_ref):
    del prev_ref
    x = x_ref[...]
    lblr = l_ref[0]
    y = jnp.dot(
        x.astype(jnp.bfloat16), w1_ref[...].astype(jnp.bfloat16),
        preferred_element_type=jnp.float32,
    )
    glo, ghi = _unpack_bf16_pair(g_ref[...])
    g = jnp.concatenate([glo, ghi], axis=1).astype(jnp.float32)
    mrow = (lblr != 0).astype(jnp.bfloat16)
    m = lax.dot_general(
        mrow, jnp.ones((1, 1), jnp.bfloat16), (((0,), (0,)), ((), ())),
        preferred_element_type=jnp.float32,
    )
    o_ref[...] = x + m * (y + g - x)


_OUT_SHAPE = jax.ShapeDtypeStruct((NTOK, H), jnp.float32)


def _NB_OFF():
    return jnp.int32(NB_SC)


def _merge_onehot(x, lbl2d, label_feature, b2d, w_merge):
    return pl.pallas_call(
        _merge_oh_body,
        grid=(NB_OH,),
        in_specs=[
            pl.BlockSpec((BT, H), lambda i: (i + _NB_OFF(), _I0())),
            pl.BlockSpec((1, 1, BT), lambda i: (i + _NB_OFF(), _I0(), _I0())),
            pl.BlockSpec((NLBL, H), lambda i: (_I0(), _I0())),
            pl.BlockSpec((H, H), lambda i: (jnp.int32(1), _I0())),
            pl.BlockSpec((H,), lambda i: (_I0(),)),
            pl.BlockSpec((H, H), lambda i: (_I0(), _I0())),
        ],
        out_specs=pl.BlockSpec((BT, H), lambda i: (i + _NB_OFF(), _I0())),
        out_shape=_OUT_SHAPE,
        scratch_shapes=[pltpu.VMEM((NLBL, H), jnp.bfloat16)],
        compiler_params=pltpu.CompilerParams(
            dimension_semantics=("parallel",)
        ),
    )(x, lbl2d, label_feature, w_merge, b2d, w_merge)


BTG = 512
NB_G = SPLIT // BTG


def _merge_gather(x, lbl2d, g, w_merge, prev):
    return pl.pallas_call(
        _merge_gather_body,
        grid=(NB_G,),
        in_specs=[
            pl.BlockSpec((BTG, H), lambda i: (i, _I0())),
            pl.BlockSpec((1, 1, BTG), lambda i: (_I0(), _I0(), i)),
            pl.BlockSpec((BTG, HP), lambda i: (i, _I0())),
            pl.BlockSpec((H, H), lambda i: (_I0(), _I0())),
            pl.BlockSpec((8, 128), lambda i: (_I0(), _I0())),
        ],
        out_specs=pl.BlockSpec((BTG, H), lambda i: (i, _I0())),
        out_shape=_OUT_SHAPE,
        input_output_aliases={4: 0},
        compiler_params=pltpu.CompilerParams(
            dimension_semantics=("parallel",)
        ),
    )(x, lbl2d, g, w_merge, prev)


def kernel(com_features, labels, label_feature, W_merge, b_merge):
    x = com_features.reshape(NTOK, H)
    lbl3d = labels.reshape(NTOK // BT, 1, BT).astype(jnp.int32)
    t_packed = _project_table(label_feature, W_merge, b_merge)
    g = _make_sc_gather()(lbl3d, t_packed)
    out = _merge_onehot(x, lbl3d, label_feature, b_merge, W_merge)
    out = _merge_gather(x, lbl3d, g, W_merge, out)
    return out.reshape(com_features.shape)
